# baseline (speedup 1.0000x reference)
import jax
import jax.numpy as jnp
from jax.experimental import pallas as pl
from jax.experimental.pallas import tpu as pltpu
from jax.experimental.pallas import tpu_sc as plsc

_VOCAB = 100000
_EMBED = 128
_BATCH = 1024

_GATHER_WINDOW = 128

_BN = 1408
_NB = 71
_MAIN = _NB * _BN
_TAIL = _VOCAB - _MAIN


def _sc_gather(emb_table, indices):
    mesh = plsc.VectorSubcoreMesh(core_axis_name="core",
                                  subcore_axis_name="subcore")
    idx2d = indices.reshape(1, _BATCH)

    @pl.kernel(
        out_type=jax.ShapeDtypeStruct((_BATCH, _EMBED), emb_table.dtype),
        mesh=mesh,
    )
    def gather_kernel(tbl_hbm, idx_hbm, out_hbm):
        def body(idx_vmem, out_vmem):
            pltpu.sync_copy(tbl_hbm.at[idx_vmem.at[0]], out_vmem)

        pltpu.emit_pipeline(
            body,
            grid=(_BATCH // _GATHER_WINDOW,),
            in_specs=[pl.BlockSpec((1, _GATHER_WINDOW),
                                   index_map=lambda i: (0, i))],
            out_specs=[pl.BlockSpec((_GATHER_WINDOW, _EMBED),
                                    index_map=lambda i: (i, 0))],
            core_axis_name=("core", "subcore"),
            dimension_semantics=(pltpu.PARALLEL,),
        )(idx_hbm, out_hbm)

    return gather_kernel(emb_table, idx2d)


def _proj_tile_body(h_ref, w_ref, b_ref, o_ref):
    h = h_ref[...].astype(jnp.bfloat16)
    w = w_ref[...].astype(jnp.bfloat16)
    acc = jax.lax.dot_general(
        h, w,
        dimension_numbers=(((1,), (1,)), ((), ())),
        preferred_element_type=jnp.float32,
    )
    o_ref[...] = acc + b_ref[...]


def _tc_project(h, W, b2d, w_tail, b_tail):
    mesh = pltpu.create_tensorcore_mesh("core", num_cores=2)

    @pl.kernel(
        out_type=[
            jax.ShapeDtypeStruct((_BATCH, _VOCAB), jnp.float32),
            jax.ShapeDtypeStruct((_BATCH, _TAIL), jnp.float32),
        ],
        mesh=mesh,
        scratch_types=[
            pltpu.VMEM((_BATCH, _EMBED), jnp.float32),
            pltpu.VMEM((_TAIL, _EMBED), jnp.float32),
            pltpu.VMEM((1, _TAIL), jnp.float32),
            pltpu.VMEM((_BATCH, _TAIL), jnp.float32),
            pltpu.SemaphoreType.DMA,
        ],
    )
    def proj_kernel(h_hbm, w_hbm, b_hbm, wt_hbm, bt_hbm, o_hbm, t_hbm,
                    h_vmem, wt_vmem, bt_vmem, acc_vmem, sem):
        pltpu.emit_pipeline(
            _proj_tile_body,
            grid=(_NB,),
            in_specs=[
                pl.BlockSpec((_BATCH, _EMBED), lambda j: (0, 0)),
                pl.BlockSpec((_BN, _EMBED), lambda j: (j, 0)),
                pl.BlockSpec((1, _BN), lambda j: (0, j)),
            ],
            out_specs=[
                pl.BlockSpec((_BATCH, _BN), lambda j: (0, j)),
            ],
            core_axis_name="core",
            dimension_semantics=(pltpu.PARALLEL,),
        )(h_hbm, w_hbm, b_hbm, o_hbm.at[:, pl.ds(0, _MAIN)])

        @pl.when(jax.lax.axis_index("core") == 0)
        def _():
            pltpu.async_copy(h_hbm, h_vmem, sem).wait()
            pltpu.async_copy(wt_hbm, wt_vmem, sem).wait()
            pltpu.async_copy(bt_hbm, bt_vmem, sem).wait()
            acc = jax.lax.dot_general(
                h_vmem[...].astype(jnp.bfloat16),
                wt_vmem[...].astype(jnp.bfloat16),
                dimension_numbers=(((1,), (1,)), ((), ())),
                preferred_element_type=jnp.float32,
            )
            acc_vmem[...] = acc + bt_vmem[...]
            pltpu.async_copy(acc_vmem, t_hbm, sem).wait()

    return proj_kernel(h, W, b2d, w_tail, b_tail)


def _patch_body(o_in, t_ref, o_blk):
    del o_in
    o_blk[:, : _TAIL] = t_ref[...]


def _patch_tail(o, tail):
    return pl.pallas_call(
        _patch_body,
        grid=(1,),
        in_specs=[
            pl.BlockSpec(memory_space=pltpu.MemorySpace.HBM),
            pl.BlockSpec((_BATCH, _TAIL), lambda i: (0, 0)),
        ],
        out_specs=pl.BlockSpec((_BATCH, 128), lambda i: (0, _VOCAB // 128)),
        out_shape=jax.ShapeDtypeStruct((_BATCH, _VOCAB), jnp.float32),
        input_output_aliases={0: 0},
    )(o, tail)


def kernel(model_in, emb_table, W, b):
    idx = model_in.astype(jnp.int32)
    h = _sc_gather(emb_table, idx)
    w_tail = W[_MAIN:]
    b_tail = b[_MAIN:].reshape(1, _TAIL)
    o, tail = _tc_project(h, W, b.reshape(1, _VOCAB), w_tail, b_tail)
    return _patch_tail(o, tail)
